# baseline (speedup 1.0000x reference)
import jax
import jax.numpy as jnp
from jax import lax
from jax.experimental import pallas as pl
from jax.experimental.pallas import tpu as pltpu
from jax.experimental.pallas import tpu_sc as plsc

_NC = 2
_NS = 16
_NW = _NC * _NS

_BATCH = 16384
_HIST = 200
_D = 32
_BPW = _BATCH // _NW


def _gather_body(tokens_hbm, table_hbm, out_hbm, idx_v, rows_v, sem):
    wid = lax.axis_index("s") * _NC + lax.axis_index("c")
    b0 = pl.multiple_of(wid * _BPW, _BPW)

    def chunk(h, carry):
        pltpu.sync_copy(tokens_hbm.at[h, pl.ds(b0, _BPW)], idx_v)
        pltpu.async_copy(table_hbm.at[idx_v], rows_v, sem).wait()
        pltpu.sync_copy(rows_v, out_hbm.at[h, pl.ds(b0, _BPW)])
        return carry

    lax.fori_loop(0, _HIST, chunk, 0)


_sc_gather = pl.kernel(
    _gather_body,
    out_type=jax.ShapeDtypeStruct((_HIST, _BATCH, _D), jnp.float32),
    mesh=plsc.VectorSubcoreMesh(core_axis_name="c", subcore_axis_name="s"),
    scratch_types=[
        pltpu.VMEM((_BPW,), jnp.int32),
        pltpu.VMEM((_BPW, _D), jnp.float32),
        pltpu.SemaphoreType.DMA,
    ],
    compiler_params=pltpu.CompilerParams(use_tc_tiling_on_sc=False),
)


@jax.jit
def kernel(tokens, embedding_weights):
    out_hbe = _sc_gather(tokens.astype(jnp.int32).T, embedding_weights)
    return jnp.transpose(out_hbe, (1, 0, 2))
